# baseline (speedup 1.0000x reference)
import functools

import jax
import jax.numpy as jnp
from jax import lax
from jax.experimental import pallas as pl
from jax.experimental.pallas import tpu as pltpu
from jax.experimental.pallas import tpu_sc as plsc

EMB = 128
D0, D1, D2 = 6, 7, 3
NROWS = D0 * D1 * D2
UNIT = 256
HALF = 128
NW = 32


def _encoder_call(E):
    nunits = E // UNIT
    full_rounds = nunits // NW
    tail = nunits - full_rounds * NW
    pairs = full_rounds // 2
    odd_round = full_rounds - pairs * 2
    mesh = plsc.VectorSubcoreMesh(core_axis_name="c", subcore_axis_name="s")

    @functools.partial(
        pl.kernel,
        out_type=jax.ShapeDtypeStruct((E, EMB), jnp.float32),
        mesh=mesh,
        scratch_types=[
            pltpu.VMEM((D0, EMB), jnp.float32),
            pltpu.VMEM((D1, EMB), jnp.float32),
            pltpu.VMEM((D2, EMB), jnp.float32),
            pltpu.VMEM((NROWS, EMB), jnp.float32),
            pltpu.VMEM_SHARED((NROWS, EMB), jnp.float32),
            pltpu.VMEM((2, 6, HALF), jnp.int32),
            pltpu.VMEM((2, 2, HALF), jnp.int32),
            pltpu.VMEM((UNIT, EMB), jnp.float32),
            pltpu.VMEM((UNIT, EMB), jnp.float32),
            pltpu.SemaphoreType.DMA,
            pltpu.SemaphoreType.DMA,
            pltpu.SemaphoreType.DMA,
            pltpu.SemaphoreType.DMA,
            pltpu.SemaphoreType.DMA,
        ],
    )
    def k(af, w0, w1, w2, out, w0_v, w1_v, w2_v, t_v, t_sh,
          i6, cb, rows0, rows1, isem, gsem0, gsem1, wsem0, wsem1):
        rows = (rows0, rows1)
        gsem = (gsem0, gsem1)
        wsem = (wsem0, wsem1)
        cid = lax.axis_index("c")
        sid = lax.axis_index("s")
        wid = sid * 2 + cid

        def fetch_idx(s, t):
            pltpu.async_copy(af.at[t], i6.at[s], isem)

        def wait_idx(s):
            pltpu.make_async_copy(af.at[0], i6.at[s], isem).wait()

        def compute(s):
            for h in range(2):
                for i in range(HALF // 16):
                    o = i * 16
                    v0 = jnp.minimum(i6[s, h, pl.ds(o, 16)], D0 - 1)
                    v1 = jnp.minimum(i6[s, 2 + h, pl.ds(o, 16)], D1 - 1)
                    v2 = jnp.minimum(i6[s, 4 + h, pl.ds(o, 16)], D2 - 1)
                    cb[s, h, pl.ds(o, 16)] = (
                        v0 * (D1 * D2) + v1 * D2 + v2)

        def gather_start(s):
            for h in range(2):
                pltpu.async_copy(
                    t_sh.at[cb.at[s, h]],
                    rows[s].at[pl.ds(h * HALF, HALF)],
                    gsem[s])

        def gather_wait(s):
            for h in range(2):
                pltpu.make_async_copy(
                    t_sh.at[cb.at[s, h]],
                    rows[s].at[pl.ds(h * HALF, HALF)],
                    gsem[s]).wait()

        def write(s, t):
            pltpu.async_copy(
                rows[s], out.at[pl.ds(t * UNIT, UNIT)], wsem[s])

        def wait_write(s):
            pltpu.make_async_copy(
                rows[s], out.at[pl.ds(0, UNIT)], wsem[s]).wait()

        fetch_idx(0, wid)

        @pl.when(sid == 0)
        def _build_table():
            pltpu.sync_copy(w0, w0_v)
            pltpu.sync_copy(w1, w1_v)
            pltpu.sync_copy(w2, w2_v)

            def row(r, carry):
                r0 = r // (D1 * D2)
                rem_ = r % (D1 * D2)
                r1 = rem_ // D2
                r2 = rem_ % D2

                def seg(si, c2):
                    o = si * 16
                    t_v[r, pl.ds(o, 16)] = (
                        w0_v[r0, pl.ds(o, 16)]
                        + w1_v[r1, pl.ds(o, 16)]
                        + w2_v[r2, pl.ds(o, 16)]
                    )
                    return c2

                lax.fori_loop(0, EMB // 16, seg, None)
                return carry

            lax.fori_loop(0, NROWS, row, None)
            pltpu.sync_copy(t_v, t_sh)

        plsc.subcore_barrier()

        def step(j, s, guard_recycle, guard_flush):
            t = j * NW + wid
            wait_idx(s)
            compute(s)

            @pl.when(j < full_rounds - 1)
            def _prefetch():
                fetch_idx(1 - s, (j + 1) * NW + wid)

            if guard_recycle is None:
                wait_write(s)
            elif guard_recycle is not False:
                @pl.when(guard_recycle)
                def _recycle():
                    wait_write(s)

            gather_start(s)

            if guard_flush is None:
                gather_wait(1 - s)
                write(1 - s, t - NW)
            elif guard_flush is not False:
                @pl.when(guard_flush)
                def _flush():
                    gather_wait(1 - s)
                    write(1 - s, t - NW)

        def do_pair(jj, carry):
            step(jj * 2, 0, jj >= 1, jj >= 1)
            step(jj * 2 + 1, 1, jj >= 1, None)
            return carry

        lax.fori_loop(0, pairs, do_pair, None)

        if odd_round:
            j = pairs * 2
            step(j, 0,
                 None if pairs >= 1 else False,
                 None if full_rounds >= 2 else False)

        if full_rounds >= 1:
            sl = (full_rounds - 1) % 2
            gather_wait(sl)
            write(sl, (full_rounds - 1) * NW + wid)
            wait_write(sl)
            if full_rounds >= 2:
                wait_write(1 - sl)

        if tail:
            @pl.when(wid < tail)
            def _tail():
                t = full_rounds * NW + wid
                fetch_idx(0, t)
                wait_idx(0)
                compute(0)
                gather_start(0)
                gather_wait(0)
                write(0, t)
                wait_write(0)

    return k


def kernel(edge_attr, W0, W1, W2):
    E = edge_attr.shape[0]
    idx = edge_attr.astype(jnp.int32)
    af = jnp.transpose(
        idx.reshape(E // UNIT, 2, HALF, 3), (0, 3, 1, 2)
    ).reshape(E // UNIT, 6, HALF)
    return _encoder_call(E)(af, W0, W1, W2)


# speedup vs baseline: 1.0574x; 1.0574x over previous
import functools

import jax
import jax.numpy as jnp
from jax import lax
from jax.experimental import pallas as pl
from jax.experimental.pallas import tpu as pltpu
from jax.experimental.pallas import tpu_sc as plsc

EMB = 128
D0, D1, D2 = 6, 7, 3
NROWS = D0 * D1 * D2
CHUNK = 128
NBUF = 3
UNROLL = 2 * NBUF
NW = 32


def _encoder_call(E):
    nchunks = E // CHUNK
    full_rounds = nchunks // NW
    tail = nchunks - full_rounds * NW
    assert full_rounds % UNROLL == 0, full_rounds
    mesh = plsc.VectorSubcoreMesh(core_axis_name="c", subcore_axis_name="s")

    @functools.partial(
        pl.kernel,
        out_type=jax.ShapeDtypeStruct((E, EMB), jnp.float32),
        mesh=mesh,
        scratch_types=[
            pltpu.VMEM((D0, EMB), jnp.float32),
            pltpu.VMEM((D1, EMB), jnp.float32),
            pltpu.VMEM((D2, EMB), jnp.float32),
            pltpu.VMEM((NROWS, EMB), jnp.float32),
            pltpu.VMEM_SHARED((NROWS, EMB), jnp.float32),
            pltpu.VMEM((2, 3, CHUNK), jnp.int32),
            pltpu.VMEM((NBUF, CHUNK), jnp.int32),
        ] + [pltpu.VMEM((CHUNK, EMB), jnp.float32)] * NBUF + [
            pltpu.SemaphoreType.DMA,
        ] + [pltpu.SemaphoreType.DMA] * (2 * NBUF),
    )
    def k(af, w0, w1, w2, out, w0_v, w1_v, w2_v, t_v, t_sh,
          i3, cb, *rows_and_sems):
        rows = rows_and_sems[:NBUF]
        isem = rows_and_sems[NBUF]
        gsems = rows_and_sems[NBUF + 1:NBUF + 1 + NBUF]
        wsems = rows_and_sems[NBUF + 1 + NBUF:]
        cid = lax.axis_index("c")
        sid = lax.axis_index("s")
        wid = sid * 2 + cid

        def fetch_idx(b, t):
            pltpu.async_copy(af.at[t], i3.at[b], isem)

        def wait_idx(b):
            pltpu.make_async_copy(af.at[0], i3.at[b], isem).wait()

        def compute(b, s):
            for i in range(CHUNK // 16):
                o = i * 16
                v0 = jnp.minimum(i3[b, 0, pl.ds(o, 16)], D0 - 1)
                v1 = jnp.minimum(i3[b, 1, pl.ds(o, 16)], D1 - 1)
                v2 = jnp.minimum(i3[b, 2, pl.ds(o, 16)], D2 - 1)
                cb[s, pl.ds(o, 16)] = v0 * (D1 * D2) + v1 * D2 + v2

        def gather_start(s):
            pltpu.async_copy(t_sh.at[cb.at[s]], rows[s], gsems[s])

        def gather_wait(s):
            pltpu.make_async_copy(
                t_sh.at[cb.at[s]], rows[s], gsems[s]).wait()

        def write(s, t):
            pltpu.async_copy(
                rows[s], out.at[pl.ds(t * CHUNK, CHUNK)], wsems[s])

        def wait_write(s):
            pltpu.make_async_copy(
                rows[s], out.at[pl.ds(0, CHUNK)], wsems[s]).wait()

        fetch_idx(0, wid)

        @pl.when(sid == 0)
        def _build_table():
            pltpu.sync_copy(w0, w0_v)
            pltpu.sync_copy(w1, w1_v)
            pltpu.sync_copy(w2, w2_v)

            def row(r, carry):
                r0 = r // (D1 * D2)
                rem_ = r % (D1 * D2)
                r1 = rem_ // D2
                r2 = rem_ % D2

                def seg(si, c2):
                    o = si * 16
                    t_v[r, pl.ds(o, 16)] = (
                        w0_v[r0, pl.ds(o, 16)]
                        + w1_v[r1, pl.ds(o, 16)]
                        + w2_v[r2, pl.ds(o, 16)]
                    )
                    return c2

                lax.fori_loop(0, EMB // 16, seg, None)
                return carry

            lax.fori_loop(0, NROWS, row, None)
            pltpu.sync_copy(t_v, t_sh)

        plsc.subcore_barrier()

        def do_group(gg, carry):
            for u in range(UNROLL):
                j = gg * UNROLL + u
                b = u % 2
                s = u % NBUF
                t = j * NW + wid
                wait_idx(b)
                compute(b, s)

                @pl.when(j < full_rounds - 1)
                def _prefetch():
                    fetch_idx(1 - b, (j + 1) * NW + wid)

                if u >= NBUF:
                    wait_write(s)
                else:
                    @pl.when(gg >= 1)
                    def _recycle():
                        wait_write(s)

                gather_start(s)

                sp = (u - 1) % NBUF
                if u >= 1:
                    gather_wait(sp)
                    write(sp, t - NW)
                else:
                    @pl.when(gg >= 1)
                    def _flush_prev():
                        gather_wait(sp)
                        write(sp, t - NW)
            return carry

        lax.fori_loop(0, full_rounds // UNROLL, do_group, None)

        sl = (full_rounds - 1) % NBUF
        gather_wait(sl)
        write(sl, (full_rounds - 1) * NW + wid)
        for s in range(NBUF):
            wait_write(s)

        if tail:
            @pl.when(wid < tail)
            def _tail():
                t = full_rounds * NW + wid
                fetch_idx(0, t)
                wait_idx(0)
                compute(0, 0)
                gather_start(0)
                gather_wait(0)
                write(0, t)
                wait_write(0)

    return k


def kernel(edge_attr, W0, W1, W2):
    E = edge_attr.shape[0]
    idx = edge_attr.astype(jnp.int32)
    af = jnp.transpose(idx.reshape(E // CHUNK, CHUNK, 3), (0, 2, 1))
    return _encoder_call(E)(af, W0, W1, W2)
